# baseline (speedup 1.0000x reference)
import jax
import jax.numpy as jnp
from jax.experimental import pallas as pl
from jax.experimental.pallas import tpu as pltpu

_BM = 256


def _fused_kernel(a_ref, b_ref, w_ref, bias_ref, out_ref, h_ref):
    @pl.when(pl.program_id(0) == 0)
    def _():
        h_ref[...] = (
            jnp.dot(b_ref[...], w_ref[...], preferred_element_type=jnp.float32)
            + bias_ref[...]
        )

    out_ref[...] = jnp.dot(
        a_ref[...], h_ref[...], preferred_element_type=jnp.float32
    )


def kernel(a, b, W, bias):
    n, k = a.shape
    d_in, d_out = W.shape
    bias2 = bias.reshape(1, d_out)
    return pl.pallas_call(
        _fused_kernel,
        grid=(n // _BM,),
        in_specs=[
            pl.BlockSpec((_BM, k), lambda i: (i, 0)),
            pl.BlockSpec((k, d_in), lambda i: (0, 0)),
            pl.BlockSpec((d_in, d_out), lambda i: (0, 0)),
            pl.BlockSpec((1, d_out), lambda i: (0, 0)),
        ],
        out_specs=pl.BlockSpec((_BM, d_out), lambda i: (i, 0)),
        out_shape=jax.ShapeDtypeStruct((n, d_out), jnp.float32),
        scratch_shapes=[pltpu.VMEM((k, d_out), jnp.float32)],
    )(a, b, W, bias2)


# speedup vs baseline: 1.0901x; 1.0901x over previous
import jax
import jax.numpy as jnp
from jax.experimental import pallas as pl
from jax.experimental.pallas import tpu as pltpu

_BM = 1024


def _fused_kernel(a_ref, b_ref, w_ref, bias_ref, out_ref, h_ref):
    @pl.when(pl.program_id(0) == 0)
    def _():
        h_ref[...] = (
            jnp.dot(b_ref[...], w_ref[...], preferred_element_type=jnp.float32)
            + bias_ref[...]
        )

    out_ref[...] = jnp.dot(
        a_ref[...], h_ref[...], preferred_element_type=jnp.float32
    )


def kernel(a, b, W, bias):
    n, k = a.shape
    d_in, d_out = W.shape
    bias2 = bias.reshape(1, d_out)
    return pl.pallas_call(
        _fused_kernel,
        grid=(n // _BM,),
        in_specs=[
            pl.BlockSpec((_BM, k), lambda i: (i, 0)),
            pl.BlockSpec((k, d_in), lambda i: (0, 0)),
            pl.BlockSpec((d_in, d_out), lambda i: (0, 0)),
            pl.BlockSpec((1, d_out), lambda i: (0, 0)),
        ],
        out_specs=pl.BlockSpec((_BM, d_out), lambda i: (i, 0)),
        out_shape=jax.ShapeDtypeStruct((n, d_out), jnp.float32),
        scratch_shapes=[pltpu.VMEM((k, d_out), jnp.float32)],
    )(a, b, W, bias2)


# speedup vs baseline: 1.1129x; 1.0209x over previous
import jax
import jax.numpy as jnp
from jax.experimental import pallas as pl
from jax.experimental.pallas import tpu as pltpu

_BK = 512


def _fused_kernel(a_ref, b_ref, w_ref, bias_ref, out_ref):
    h = (
        jnp.dot(b_ref[...], w_ref[...], preferred_element_type=jnp.float32)
        + bias_ref[...]
    )
    part = jnp.dot(a_ref[...], h, preferred_element_type=jnp.float32)

    @pl.when(pl.program_id(0) == 0)
    def _():
        out_ref[...] = part

    @pl.when(pl.program_id(0) != 0)
    def _():
        out_ref[...] += part


def kernel(a, b, W, bias):
    n, k = a.shape
    d_in, d_out = W.shape
    bias2 = bias.reshape(1, d_out)
    return pl.pallas_call(
        _fused_kernel,
        grid=(k // _BK,),
        in_specs=[
            pl.BlockSpec((n, _BK), lambda i: (0, i)),
            pl.BlockSpec((_BK, d_in), lambda i: (i, 0)),
            pl.BlockSpec((d_in, d_out), lambda i: (0, 0)),
            pl.BlockSpec((1, d_out), lambda i: (0, 0)),
        ],
        out_specs=pl.BlockSpec((n, d_out), lambda i: (0, 0)),
        out_shape=jax.ShapeDtypeStruct((n, d_out), jnp.float32),
    )(a, b, W, bias2)


# speedup vs baseline: 1.1144x; 1.0013x over previous
import jax
import jax.numpy as jnp
from jax.experimental import pallas as pl
from jax.experimental.pallas import tpu as pltpu

_BM = 512
_SPLITS = 4


def _fused_kernel(a0_ref, a1_ref, a2_ref, a3_ref, b_ref, w_ref, bias_ref,
                  out_ref, h_ref):
    @pl.when(pl.program_id(0) == 0)
    def _():
        h_ref[...] = (
            jnp.dot(b_ref[...], w_ref[...], preferred_element_type=jnp.float32)
            + bias_ref[...]
        )

    kq = a0_ref.shape[1]
    acc = jnp.dot(a0_ref[...], h_ref[0 * kq:1 * kq, :],
                  preferred_element_type=jnp.float32)
    acc += jnp.dot(a1_ref[...], h_ref[1 * kq:2 * kq, :],
                   preferred_element_type=jnp.float32)
    acc += jnp.dot(a2_ref[...], h_ref[2 * kq:3 * kq, :],
                   preferred_element_type=jnp.float32)
    acc += jnp.dot(a3_ref[...], h_ref[3 * kq:4 * kq, :],
                   preferred_element_type=jnp.float32)
    out_ref[...] = acc


def kernel(a, b, W, bias):
    n, k = a.shape
    d_in, d_out = W.shape
    kq = k // _SPLITS
    bias2 = bias.reshape(1, d_out)

    def a_spec(s):
        return pl.BlockSpec((_BM, kq), lambda i, s=s: (i, s))

    return pl.pallas_call(
        _fused_kernel,
        grid=(n // _BM,),
        in_specs=[
            a_spec(0), a_spec(1), a_spec(2), a_spec(3),
            pl.BlockSpec((k, d_in), lambda i: (0, 0)),
            pl.BlockSpec((d_in, d_out), lambda i: (0, 0)),
            pl.BlockSpec((1, d_out), lambda i: (0, 0)),
        ],
        out_specs=pl.BlockSpec((_BM, d_out), lambda i: (i, 0)),
        out_shape=jax.ShapeDtypeStruct((n, d_out), jnp.float32),
        scratch_shapes=[pltpu.VMEM((k, d_out), jnp.float32)],
    )(a, a, a, a, b, W, bias2)
